# Initial kernel scaffold; baseline (speedup 1.0000x reference)
import jax
import jax.numpy as jnp
from jax.experimental import pallas as pl


def kernel(input_ids, token_type_ids, word_emb, pos_emb, tok_emb, gamma, beta):
    raise NotImplementedError("write your pallas kernel here")


# speedup vs baseline: 2.0788x; 2.0788x over previous
import functools

import jax
import jax.numpy as jnp
from jax import lax
from jax.experimental import pallas as pl
from jax.experimental.pallas import tpu as pltpu
from jax.experimental.pallas import tpu_sc as plsc

EPS_LN = 1e-12

_NC = 2
_NS = 16
_NW = _NC * _NS
_GW = 16


def _sc_gather(word_emb, flat_ids):
    n = flat_ids.shape[0]
    _, d = word_emb.shape
    b_per_w = n // _NW
    nchunks = b_per_w // _GW
    mesh = plsc.VectorSubcoreMesh(core_axis_name="c", subcore_axis_name="s")

    @functools.partial(
        pl.kernel,
        mesh=mesh,
        out_type=jax.ShapeDtypeStruct((n, d), word_emb.dtype),
        scratch_types=[
            pltpu.VMEM((b_per_w,), jnp.int32),
            pltpu.VMEM((_GW, d), word_emb.dtype),
            pltpu.SemaphoreType.DMA,
        ],
    )
    def gather_kernel(table_hbm, idx_hbm, out_hbm, idx_v, rows_v, sem):
        wid = lax.axis_index("s") * _NC + lax.axis_index("c")
        base = wid * b_per_w
        pltpu.sync_copy(idx_hbm.at[pl.ds(base, b_per_w)], idx_v)

        @pl.loop(0, nchunks)
        def _(c):
            off = c * _GW
            pltpu.async_copy(
                table_hbm.at[idx_v.at[pl.ds(off, _GW)]], rows_v, sem
            ).wait()
            pltpu.sync_copy(rows_v, out_hbm.at[pl.ds(base + off, _GW)])

    return gather_kernel(word_emb, flat_ids)


def _ln_body(g_ref, pos_ref, tt_ref, tok_ref, gam_ref, bet_ref, o_ref):
    e = g_ref[...] + pos_ref[...]
    w = tt_ref[...]
    e = e + (tok_ref[0:1, :] * (1.0 - w) + tok_ref[1:2, :] * w)
    mu = jnp.mean(e, axis=1, keepdims=True)
    dlt = e - mu
    var = jnp.mean(dlt * dlt, axis=1, keepdims=True)
    o_ref[...] = dlt * lax.rsqrt(var + EPS_LN) * gam_ref[...] + bet_ref[...]


def _tc_ln(gathered, tt_w, pos_emb, tok_emb, gamma2d, beta2d, b, s):
    n, h = gathered.shape
    tb = 256
    n_s = s // tb
    grid = (n_s, b)
    return pl.pallas_call(
        _ln_body,
        grid=grid,
        in_specs=[
            pl.BlockSpec((tb, h), lambda i, bb: (bb * n_s + i, 0)),
            pl.BlockSpec((tb, h), lambda i, bb: (i, 0)),
            pl.BlockSpec((tb, 1), lambda i, bb: (bb * n_s + i, 0)),
            pl.BlockSpec(tok_emb.shape, lambda i, bb: (0, 0)),
            pl.BlockSpec((1, h), lambda i, bb: (0, 0)),
            pl.BlockSpec((1, h), lambda i, bb: (0, 0)),
        ],
        out_specs=pl.BlockSpec((tb, h), lambda i, bb: (bb * n_s + i, 0)),
        out_shape=jax.ShapeDtypeStruct((n, h), jnp.float32),
    )(gathered, pos_emb, tt_w, tok_emb, gamma2d, beta2d)


def kernel(input_ids, token_type_ids, word_emb, pos_emb, tok_emb, gamma, beta):
    b, s = input_ids.shape
    h = word_emb.shape[1]
    flat_ids = input_ids.reshape(-1).astype(jnp.int32)
    gathered = _sc_gather(word_emb, flat_ids)
    tt_w = token_type_ids.reshape(-1, 1).astype(jnp.float32)
    out = _tc_ln(
        gathered,
        tt_w,
        pos_emb[:s],
        tok_emb,
        gamma.reshape(1, -1),
        beta.reshape(1, -1),
        b,
        s,
    )
    return out.reshape(b, s, h)
